# baseline (speedup 1.0000x reference)
import functools

import jax
import jax.numpy as jnp
from jax import lax
from jax.experimental import pallas as pl
from jax.experimental.pallas import tpu as pltpu
from jax.experimental.pallas import tpu_sc as plsc

_NC = 2
_NS = 16


def _sc_half(active, src, dst, nf_half, ef, n_nodes, de):
    e_total = src.shape[0]
    dh = nf_half.shape[1]
    load_ef = ef is not None
    ept = e_total // _NS
    ch = 80
    sb = 5
    rows_pt = ept // ch
    nsc = rows_pt // sb
    rpt = (n_nodes // (8 * _NS)) * 8
    rem = n_nodes - _NS * rpt
    rem0 = _NS * rpt
    nst = 6
    spt = rpt // nst

    mesh = plsc.VectorSubcoreMesh(
        core_axis_name="c", subcore_axis_name="s",
        num_cores=_NC, num_subcores=_NS)

    @functools.partial(
        pl.kernel,
        out_type=[
            jax.ShapeDtypeStruct((n_nodes, dh), jnp.float32),
            jax.ShapeDtypeStruct((n_nodes, de), jnp.float32),
        ],
        mesh=mesh,
        compiler_params=pltpu.CompilerParams(use_tc_tiling_on_sc=False),
        scratch_types=[
            pltpu.VMEM_SHARED((n_nodes, dh), jnp.float32),
            pltpu.VMEM_SHARED((n_nodes, de), jnp.float32),
            pltpu.VMEM((2, sb * ch), jnp.int32),
            pltpu.VMEM((2, sb * ch), jnp.int32),
            pltpu.VMEM((2, sb, ch), jnp.int32),
            pltpu.VMEM((2, sb, ch, dh), jnp.float32),
            pltpu.VMEM((2, sb * ch, de), jnp.float32),
            pltpu.VMEM((ch, de), jnp.float32),
            pltpu.VMEM((spt, dh), jnp.float32),
            pltpu.VMEM((spt, de), jnp.float32),
            pltpu.VMEM((rem, dh), jnp.float32),
            pltpu.VMEM((rem, de), jnp.float32),
            pltpu.SemaphoreType.DMA,
            pltpu.SemaphoreType.DMA,
            pltpu.SemaphoreType.DMA,
            pltpu.SemaphoreType.DMA,
            pltpu.SemaphoreType.DMA,
        ],
    )
    def sc_kernel(src_h, dst_h, nf_h, ef_h, za_h, zb_h,
                  a_out, bd_out,
                  a_sh, bd_sh,
                  src_i, dst_i, dst_s, rows_v, val_v, ones_v, sta, stb,
                  exa, exb,
                  sem_i0, sem_i1, sem_g, sem_s0, sem_s1):
        cid = lax.axis_index("c")
        sid = lax.axis_index("s")

        r0 = sid * rpt
        rowbase = sid * rows_pt

        def init_accum():
            pltpu.sync_copy(za_h, sta)
            pltpu.sync_copy(zb_h, stb)
            for k in range(nst):
                pltpu.sync_copy(sta, a_sh.at[pl.ds(r0 + k * spt, spt)])
                pltpu.sync_copy(stb, bd_sh.at[pl.ds(r0 + k * spt, spt)])
            if rem:
                @pl.when(sid == 0)
                def _zero_rem():
                    pltpu.sync_copy(za_h.at[pl.ds(0, rem)], exa)
                    pltpu.sync_copy(zb_h.at[pl.ds(0, rem)], exb)
                    pltpu.sync_copy(exa, a_sh.at[pl.ds(rem0, rem)])
                    pltpu.sync_copy(exb, bd_sh.at[pl.ds(rem0, rem)])

        def run_edges():
            if not load_ef:
                onehot = jnp.where(lax.iota(jnp.int32, de) == 0,
                                   jnp.float32(1.0), jnp.float32(0.0))

                def init_ones(i, carry):
                    ones_v[i, :] = onehot
                    return carry
                lax.fori_loop(0, ch, init_ones, 0)

            sem_i = (sem_i0, sem_i1)
            sem_s = (sem_s0, sem_s1)

            def idx_copies(row0, p, make_only):
                mk = pltpu.make_async_copy
                e0 = pl.multiple_of(row0 * ch, 8)
                ds = [mk(src_h.at[pl.ds(e0, sb * ch)], src_i.at[p],
                         sem_i[p]),
                      mk(dst_h.at[pl.ds(e0, sb * ch)], dst_i.at[p],
                         sem_i[p])]
                if not make_only:
                    for d in ds:
                        d.start()
                return ds

            def scatter_copies(p, make_only):
                mk = pltpu.make_async_copy
                ds = []
                for b in range(sb):
                    ds.append(mk(rows_v.at[p, b], a_sh.at[dst_s.at[p, b]],
                                 sem_s[p]))
                    vsrc = (val_v.at[p, pl.ds(b * ch, ch)] if load_ef
                            else ones_v)
                    ds.append(mk(vsrc, bd_sh.at[dst_s.at[p, b]], sem_s[p]))
                if not make_only:
                    for d in ds:
                        d.start(add=True)
                return ds

            idx_copies(rowbase, 0, False)
            idx_copies(rowbase + sb, 1, False)

            def section(s, p):
                row0 = rowbase + s * sb
                @pl.when(s >= 2)
                def _drain_scatters():
                    for d in scatter_copies(p, True):
                        d.wait()
                for d in idx_copies(row0, p, True):
                    d.wait()
                for b in range(sb):
                    for j in range(ch // 16):
                        dst_s[p, b, pl.ds(j * 16, 16)] = (
                            dst_i[p, pl.ds(b * ch + j * 16, 16)])
                gd = [pltpu.async_copy(
                          nf_h.at[src_i.at[p, pl.ds(b * ch, ch)]],
                          rows_v.at[p, b], sem_g)
                      for b in range(sb)]
                if load_ef:
                    gd.append(pltpu.async_copy(
                        ef_h.at[pl.ds(row0 * ch, sb * ch)], val_v.at[p],
                        sem_g))
                @pl.when(s < nsc - 1)
                def _prefetch():
                    idx_copies(row0 + sb, 1 - p, False)

                for d in gd:
                    d.wait()
                scatter_copies(p, False)

            def pairbody(k, carry):
                section(2 * k, 0)
                section(2 * k + 1, 1)
                return carry
            lax.fori_loop(0, nsc // 2, pairbody, 0)
            for p in range(2):
                for d in scatter_copies(p, True):
                    d.wait()

        def writeout():
            for k in range(nst):
                rk = r0 + k * spt
                pltpu.sync_copy(a_sh.at[pl.ds(rk, spt)], sta)
                pltpu.sync_copy(sta, a_out.at[pl.ds(rk, spt)])
                pltpu.sync_copy(bd_sh.at[pl.ds(rk, spt)], stb)
                pltpu.sync_copy(stb, bd_out.at[pl.ds(rk, spt)])
            if rem:
                @pl.when(sid == 0)
                def _write_rem():
                    pltpu.sync_copy(a_sh.at[pl.ds(rem0, rem)], exa)
                    pltpu.sync_copy(exa, a_out.at[pl.ds(rem0, rem)])
                    pltpu.sync_copy(bd_sh.at[pl.ds(rem0, rem)], exb)
                    pltpu.sync_copy(exb, bd_out.at[pl.ds(rem0, rem)])

        @pl.when(cid == active)
        def _active_core():
            init_accum()
            plsc.subcore_barrier()
            run_edges()
            plsc.subcore_barrier()
            writeout()

    za = jnp.zeros((spt, dh), jnp.float32)
    zb = jnp.zeros((spt, de), jnp.float32)
    ef_arg = ef if load_ef else jnp.zeros((8, de), jnp.float32)
    return sc_kernel(src, dst, nf_half, ef_arg, za, zb)


def _tc_finish_body(a0_ref, a1_ref, b_ref, d_ref, nf_ref, wmh_ref, wme_ref,
                    wa1_ref, wa2_ref, bm_ref, ba_ref, o_ref):
    hi = jax.lax.Precision.HIGHEST
    dh = a0_ref.shape[1]
    deg = jnp.sum(d_ref[...], axis=1, keepdims=True)
    msum = (jnp.dot(a0_ref[...], wmh_ref[:dh], precision=hi,
                    preferred_element_type=jnp.float32)
            + jnp.dot(a1_ref[...], wmh_ref[dh:], precision=hi,
                      preferred_element_type=jnp.float32)
            + jnp.dot(b_ref[...], wme_ref[...], precision=hi,
                      preferred_element_type=jnp.float32)
            + deg * bm_ref[...])
    h_neigh = msum / jnp.maximum(deg, 1.0)
    h = (jnp.dot(nf_ref[...], wa1_ref[...], precision=hi,
                 preferred_element_type=jnp.float32)
         + jnp.dot(h_neigh, wa2_ref[...], precision=hi,
                   preferred_element_type=jnp.float32)
         + ba_ref[...])
    o_ref[...] = jnp.maximum(h, 0.0)


def _tc_finish(a0, a1, b_part, d_part, nf, wmh_t, wme_t, wa1_t, wa2_t,
               b_msg, b_apply, n_nodes):
    din = nf.shape[1]
    dh = a0.shape[1]
    de = b_part.shape[1]
    dout = wmh_t.shape[1]
    rb = 1000
    grid = (n_nodes // rb,)
    return pl.pallas_call(
        _tc_finish_body,
        grid=grid,
        in_specs=[
            pl.BlockSpec((rb, dh), lambda i: (i, 0)),
            pl.BlockSpec((rb, dh), lambda i: (i, 0)),
            pl.BlockSpec((rb, de), lambda i: (i, 0)),
            pl.BlockSpec((rb, de), lambda i: (i, 0)),
            pl.BlockSpec((rb, din), lambda i: (i, 0)),
            pl.BlockSpec((din, dout), lambda i: (0, 0)),
            pl.BlockSpec((de, dout), lambda i: (0, 0)),
            pl.BlockSpec((din, dout), lambda i: (0, 0)),
            pl.BlockSpec((dout, dout), lambda i: (0, 0)),
            pl.BlockSpec((1, dout), lambda i: (0, 0)),
            pl.BlockSpec((1, dout), lambda i: (0, 0)),
        ],
        out_specs=pl.BlockSpec((rb, dout), lambda i: (i, 0)),
        out_shape=jax.ShapeDtypeStruct((n_nodes, dout), jnp.float32),
    )(a0, a1, b_part, d_part, nf, wmh_t, wme_t, wa1_t, wa2_t, b_msg, b_apply)


def kernel(nfeats, efeats, edge_index, W_msg, b_msg, W_apply, b_apply):
    n_nodes = nfeats.shape[0]
    din = nfeats.shape[2]
    de = efeats.shape[2]
    dout = W_msg.shape[0]
    dh = din // 2

    nf = nfeats.reshape(n_nodes, din)
    ef = efeats.reshape(efeats.shape[0], de)
    src = edge_index[0]
    dst = edge_index[1]

    wmh_t = W_msg[:, :din].T
    wme_t = W_msg[:, din:].T
    wa1_t = W_apply[:, :din].T
    wa2_t = W_apply[:, din:].T

    a0, b_part = _sc_half(0, src, dst, nf[:, :dh], ef, n_nodes, de)
    a1, d_part = _sc_half(1, src, dst, nf[:, dh:], None, n_nodes, de)
    out = _tc_finish(a0, a1, b_part, d_part, nf, wmh_t, wme_t, wa1_t, wa2_t,
                     b_msg.reshape(1, dout), b_apply.reshape(1, dout), n_nodes)
    return out.reshape(n_nodes, 1, dout)


# speedup vs baseline: 1.0630x; 1.0630x over previous
import functools

import jax
import jax.numpy as jnp
from jax import lax
from jax.experimental import pallas as pl
from jax.experimental.pallas import tpu as pltpu
from jax.experimental.pallas import tpu_sc as plsc

_NC = 2
_NS = 16


def _sc_accumulate(src, dst, nf0, nf1, ef, n_nodes):
    e_total = src.shape[0]
    dh = nf0.shape[1]
    de = ef.shape[1]
    ept = e_total // _NS
    ch = 80
    sb = 5
    rows_pt = ept // ch
    nsc = rows_pt // sb
    rpt = (n_nodes // (8 * _NS)) * 8
    rem = n_nodes - _NS * rpt
    rem0 = _NS * rpt
    nst = 6
    spt = rpt // nst

    mesh = plsc.VectorSubcoreMesh(
        core_axis_name="c", subcore_axis_name="s",
        num_cores=_NC, num_subcores=_NS)

    @functools.partial(
        pl.kernel,
        out_type=[
            jax.ShapeDtypeStruct((n_nodes, dh), jnp.float32),
            jax.ShapeDtypeStruct((n_nodes, dh), jnp.float32),
            jax.ShapeDtypeStruct((n_nodes, de), jnp.float32),
            jax.ShapeDtypeStruct((n_nodes, de), jnp.float32),
        ],
        mesh=mesh,
        compiler_params=pltpu.CompilerParams(use_tc_tiling_on_sc=False),
        scratch_types=[
            pltpu.VMEM_SHARED((n_nodes, dh), jnp.float32),
            pltpu.VMEM_SHARED((n_nodes, de), jnp.float32),
            pltpu.VMEM((2, sb * ch), jnp.int32),
            pltpu.VMEM((2, sb * ch), jnp.int32),
            pltpu.VMEM((2, sb, ch), jnp.int32),
            pltpu.VMEM((2, sb, ch, dh), jnp.float32),
            pltpu.VMEM((2, sb * ch, de), jnp.float32),
            pltpu.VMEM((ch, de), jnp.float32),
            pltpu.VMEM((spt, dh), jnp.float32),
            pltpu.VMEM((spt, de), jnp.float32),
            pltpu.VMEM((rem, dh), jnp.float32),
            pltpu.VMEM((rem, de), jnp.float32),
            pltpu.SemaphoreType.DMA,
            pltpu.SemaphoreType.DMA,
            pltpu.SemaphoreType.DMA,
            pltpu.SemaphoreType.DMA,
            pltpu.SemaphoreType.DMA,
        ],
    )
    def sc_kernel(src_h, dst_h, nf0_h, nf1_h, ef_h, za_h, zb_h,
                  a0_out, a1_out, b_out, d_out,
                  a_sh, bd_sh,
                  src_i, dst_i, dst_s, rows_v, val_v, ones_v, sta, stb,
                  exa, exb,
                  sem_i0, sem_i1, sem_g, sem_s0, sem_s1):
        cid = lax.axis_index("c")
        sid = lax.axis_index("s")

        r0 = sid * rpt
        rowbase = sid * rows_pt

        def init_accum():
            pltpu.sync_copy(za_h, sta)
            pltpu.sync_copy(zb_h, stb)
            for k in range(nst):
                pltpu.sync_copy(sta, a_sh.at[pl.ds(r0 + k * spt, spt)])
                pltpu.sync_copy(stb, bd_sh.at[pl.ds(r0 + k * spt, spt)])
            if rem:
                @pl.when(sid == 0)
                def _zero_rem():
                    pltpu.sync_copy(za_h.at[pl.ds(0, rem)], exa)
                    pltpu.sync_copy(zb_h.at[pl.ds(0, rem)], exb)
                    pltpu.sync_copy(exa, a_sh.at[pl.ds(rem0, rem)])
                    pltpu.sync_copy(exb, bd_sh.at[pl.ds(rem0, rem)])

        def run_edges(nf_h, load_ef):
            if not load_ef:
                onehot = jnp.where(lax.iota(jnp.int32, de) == 0,
                                   jnp.float32(1.0), jnp.float32(0.0))

                def init_ones(i, carry):
                    ones_v[i, :] = onehot
                    return carry
                lax.fori_loop(0, ch, init_ones, 0)

            sem_i = (sem_i0, sem_i1)
            sem_s = (sem_s0, sem_s1)

            def idx_copies(row0, p, make_only):
                mk = pltpu.make_async_copy
                e0 = pl.multiple_of(row0 * ch, 8)
                ds = [mk(src_h.at[pl.ds(e0, sb * ch)], src_i.at[p],
                         sem_i[p]),
                      mk(dst_h.at[pl.ds(e0, sb * ch)], dst_i.at[p],
                         sem_i[p])]
                if not make_only:
                    for d in ds:
                        d.start()
                return ds

            def scatter_copies(p, make_only):
                mk = pltpu.make_async_copy
                ds = []
                for b in range(sb):
                    ds.append(mk(rows_v.at[p, b], a_sh.at[dst_s.at[p, b]],
                                 sem_s[p]))
                    vsrc = (val_v.at[p, pl.ds(b * ch, ch)] if load_ef
                            else ones_v)
                    ds.append(mk(vsrc, bd_sh.at[dst_s.at[p, b]], sem_s[p]))
                if not make_only:
                    for d in ds:
                        d.start(add=True)
                return ds

            idx_copies(rowbase, 0, False)
            idx_copies(rowbase + sb, 1, False)

            def section(s, p):
                row0 = rowbase + s * sb
                @pl.when(s >= 2)
                def _drain_scatters():
                    for d in scatter_copies(p, True):
                        d.wait()
                for d in idx_copies(row0, p, True):
                    d.wait()
                for b in range(sb):
                    for j in range(ch // 16):
                        dst_s[p, b, pl.ds(j * 16, 16)] = (
                            dst_i[p, pl.ds(b * ch + j * 16, 16)])
                gd = [pltpu.async_copy(
                          nf_h.at[src_i.at[p, pl.ds(b * ch, ch)]],
                          rows_v.at[p, b], sem_g)
                      for b in range(sb)]
                if load_ef:
                    gd.append(pltpu.async_copy(
                        ef_h.at[pl.ds(row0 * ch, sb * ch)], val_v.at[p],
                        sem_g))
                @pl.when(s < nsc - 1)
                def _prefetch():
                    idx_copies(row0 + sb, 1 - p, False)

                for d in gd:
                    d.wait()
                scatter_copies(p, False)

            def pairbody(k, carry):
                section(2 * k, 0)
                section(2 * k + 1, 1)
                return carry
            lax.fori_loop(0, nsc // 2, pairbody, 0)
            for p in range(2):
                for d in scatter_copies(p, True):
                    d.wait()

        def writeout(a_out, bd_out):
            for k in range(nst):
                rk = r0 + k * spt
                pltpu.sync_copy(a_sh.at[pl.ds(rk, spt)], sta)
                pltpu.sync_copy(sta, a_out.at[pl.ds(rk, spt)])
                pltpu.sync_copy(bd_sh.at[pl.ds(rk, spt)], stb)
                pltpu.sync_copy(stb, bd_out.at[pl.ds(rk, spt)])
            if rem:
                @pl.when(sid == 0)
                def _write_rem():
                    pltpu.sync_copy(a_sh.at[pl.ds(rem0, rem)], exa)
                    pltpu.sync_copy(exa, a_out.at[pl.ds(rem0, rem)])
                    pltpu.sync_copy(bd_sh.at[pl.ds(rem0, rem)], exb)
                    pltpu.sync_copy(exb, bd_out.at[pl.ds(rem0, rem)])

        init_accum()
        plsc.subcore_barrier()

        @pl.when(cid == 0)
        def _core0():
            run_edges(nf0_h, True)

        @pl.when(cid == 1)
        def _core1():
            run_edges(nf1_h, False)

        plsc.subcore_barrier()

        @pl.when(cid == 0)
        def _wr0():
            writeout(a0_out, b_out)

        @pl.when(cid == 1)
        def _wr1():
            writeout(a1_out, d_out)

    za = jnp.zeros((spt, dh), jnp.float32)
    zb = jnp.zeros((spt, de), jnp.float32)
    return sc_kernel(src, dst, nf0, nf1, ef, za, zb)


def _tc_finish_body(a0_ref, a1_ref, b_ref, d_ref, nf_ref, wmh_ref, wme_ref,
                    wa1_ref, wa2_ref, bm_ref, ba_ref, o_ref):
    hi = jax.lax.Precision.HIGHEST
    dh = a0_ref.shape[1]
    deg = jnp.sum(d_ref[...], axis=1, keepdims=True)
    msum = (jnp.dot(a0_ref[...], wmh_ref[:dh], precision=hi,
                    preferred_element_type=jnp.float32)
            + jnp.dot(a1_ref[...], wmh_ref[dh:], precision=hi,
                      preferred_element_type=jnp.float32)
            + jnp.dot(b_ref[...], wme_ref[...], precision=hi,
                      preferred_element_type=jnp.float32)
            + deg * bm_ref[...])
    h_neigh = msum / jnp.maximum(deg, 1.0)
    h = (jnp.dot(nf_ref[...], wa1_ref[...], precision=hi,
                 preferred_element_type=jnp.float32)
         + jnp.dot(h_neigh, wa2_ref[...], precision=hi,
                   preferred_element_type=jnp.float32)
         + ba_ref[...])
    o_ref[...] = jnp.maximum(h, 0.0)


def _tc_finish(a0, a1, b_part, d_part, nf, wmh_t, wme_t, wa1_t, wa2_t,
               b_msg, b_apply, n_nodes):
    din = nf.shape[1]
    dh = a0.shape[1]
    de = b_part.shape[1]
    dout = wmh_t.shape[1]
    rb = 1000
    grid = (n_nodes // rb,)
    return pl.pallas_call(
        _tc_finish_body,
        grid=grid,
        in_specs=[
            pl.BlockSpec((rb, dh), lambda i: (i, 0)),
            pl.BlockSpec((rb, dh), lambda i: (i, 0)),
            pl.BlockSpec((rb, de), lambda i: (i, 0)),
            pl.BlockSpec((rb, de), lambda i: (i, 0)),
            pl.BlockSpec((rb, din), lambda i: (i, 0)),
            pl.BlockSpec((din, dout), lambda i: (0, 0)),
            pl.BlockSpec((de, dout), lambda i: (0, 0)),
            pl.BlockSpec((din, dout), lambda i: (0, 0)),
            pl.BlockSpec((dout, dout), lambda i: (0, 0)),
            pl.BlockSpec((1, dout), lambda i: (0, 0)),
            pl.BlockSpec((1, dout), lambda i: (0, 0)),
        ],
        out_specs=pl.BlockSpec((rb, dout), lambda i: (i, 0)),
        out_shape=jax.ShapeDtypeStruct((n_nodes, dout), jnp.float32),
    )(a0, a1, b_part, d_part, nf, wmh_t, wme_t, wa1_t, wa2_t, b_msg, b_apply)


def kernel(nfeats, efeats, edge_index, W_msg, b_msg, W_apply, b_apply):
    n_nodes = nfeats.shape[0]
    din = nfeats.shape[2]
    de = efeats.shape[2]
    dout = W_msg.shape[0]
    dh = din // 2

    nf = nfeats.reshape(n_nodes, din)
    ef = efeats.reshape(efeats.shape[0], de)
    src = edge_index[0]
    dst = edge_index[1]

    wmh_t = W_msg[:, :din].T
    wme_t = W_msg[:, din:].T
    wa1_t = W_apply[:, :din].T
    wa2_t = W_apply[:, din:].T

    a0, a1, b_part, d_part = _sc_accumulate(
        src, dst, nf[:, :dh], nf[:, dh:], ef, n_nodes)
    out = _tc_finish(a0, a1, b_part, d_part, nf, wmh_t, wme_t, wa1_t, wa2_t,
                     b_msg.reshape(1, dout), b_apply.reshape(1, dout), n_nodes)
    return out.reshape(n_nodes, 1, dout)
